# baseline (speedup 1.0000x reference)
import functools

import jax
import jax.numpy as jnp
from jax import lax
from jax.experimental import pallas as pl
from jax.experimental.pallas import tpu as pltpu
from jax.experimental.pallas import tpu_sc as plsc

NC = 2
NS = 16
LANES = 16
CHUNK = 128


def _fill_vmem_zeros(ref, rows, cols):
    per_row = cols // LANES

    def body(i, carry):
        r = i // per_row
        j = i % per_row
        ref[r, pl.ds(j * LANES, LANES)] = jnp.zeros((LANES,), jnp.float32)
        return carry

    lax.fori_loop(0, rows * per_row, body, None)


@functools.lru_cache(maxsize=None)
def _make_sc_kernels(n_pad, e_pad, d, q0, q1):
    rows_per_sub = n_pad // NS
    edges_per_worker = e_pad // (NC * NS)
    n_chunks = edges_per_worker // CHUNK
    mesh = plsc.VectorSubcoreMesh(core_axis_name="c", subcore_axis_name="s")

    @functools.partial(
        pl.kernel,
        out_type=jax.ShapeDtypeStruct((NC, n_pad), jnp.float32),
        mesh=mesh,
        scratch_types=[
            pltpu.VMEM((n_chunks, CHUNK), jnp.int32),
            pltpu.VMEM((CHUNK,), jnp.float32),
            pltpu.VMEM((rows_per_sub,), jnp.float32),
            pltpu.VMEM_SHARED((n_pad,), jnp.float32),
            pltpu.SemaphoreType.DMA,
        ],
    )
    def deg_kernel(dst_hbm, out_hbm, idx_v, ones_v, zbuf_v, acc_s, sem):
        c = lax.axis_index("c")
        s = lax.axis_index("s")
        wid = c * NS + s

        def fill_ones(i, carry):
            ones_v[pl.ds(i * LANES, LANES)] = jnp.ones((LANES,), jnp.float32)
            return carry

        lax.fori_loop(0, CHUNK // LANES, fill_ones, None)

        def fill_zeros(i, carry):
            zbuf_v[pl.ds(i * LANES, LANES)] = jnp.zeros((LANES,), jnp.float32)
            return carry

        lax.fori_loop(0, rows_per_sub // LANES, fill_zeros, None)
        pltpu.sync_copy(zbuf_v, acc_s.at[pl.ds(s * rows_per_sub, rows_per_sub)])
        pltpu.sync_copy(dst_hbm.at[wid], idx_v)
        plsc.subcore_barrier()

        kgrp = 8

        def body(g, carry):
            descs = [
                pltpu.async_copy(
                    ones_v, acc_s.at[idx_v.at[g * kgrp + b]], sem, add=True)
                for b in range(kgrp)
            ]
            for dsc in descs:
                dsc.wait()
            return carry

        lax.fori_loop(0, n_chunks // kgrp, body, None)
        plsc.subcore_barrier()
        pltpu.sync_copy(
            acc_s.at[pl.ds(s * rows_per_sub, rows_per_sub)],
            out_hbm.at[c, pl.ds(s * rows_per_sub, rows_per_sub)],
        )

    pchunk = 128
    wchunks = 4
    nbuf = 2

    @functools.partial(
        pl.kernel,
        out_type=jax.ShapeDtypeStruct((NC, n_pad, d), jnp.float32),
        mesh=mesh,
        scratch_types=[
            pltpu.VMEM((2, wchunks, pchunk), jnp.int32),
            pltpu.VMEM((2, wchunks, pchunk), jnp.int32),
            pltpu.VMEM((nbuf, pchunk, d), jnp.float32),
            pltpu.VMEM_SHARED((n_pad, d), jnp.float32),
            pltpu.SemaphoreType.DMA,
            pltpu.SemaphoreType.DMA,
        ],
    )
    def prop_kernel(hs_hbm, src_hbm, dst_hbm, out_hbm,
                    si_v, di_v, rows_v, acc_s, sem_g, sem_s):
        c = lax.axis_index("c")
        s = lax.axis_index("s")
        qw = jnp.where(c == 0, q0, q1)
        wbase = c * (NS * q0) + s * qw
        nchunk_w = qw * wchunks

        _fill_vmem_zeros(rows_v.at[0], pchunk, d)

        def zero_acc(k, carry):
            pltpu.sync_copy(
                rows_v.at[0],
                acc_s.at[pl.ds(s * rows_per_sub + k * pchunk, pchunk)])
            return carry

        lax.fori_loop(0, rows_per_sub // pchunk, zero_acc, None)
        pltpu.sync_copy(src_hbm.at[wbase], si_v.at[0])
        pltpu.sync_copy(dst_hbm.at[wbase], di_v.at[0])
        plsc.subcore_barrier()

        def fire_gather(ti, buf):
            pltpu.async_copy(
                hs_hbm.at[si_v.at[lax.rem(ti // wchunks, 2),
                                  lax.rem(ti, wchunks)]],
                rows_v.at[buf], sem_g)

        def wait_gather(buf):
            pltpu.make_async_copy(
                hs_hbm.at[si_v.at[0, 0]], rows_v.at[buf], sem_g).wait()

        def fire_scatter(ti, buf):
            pltpu.async_copy(
                rows_v.at[buf],
                acc_s.at[di_v.at[lax.rem(ti // wchunks, 2),
                                 lax.rem(ti, wchunks)]],
                sem_s, add=True)

        def wait_scatter(buf):
            pltpu.make_async_copy(
                rows_v.at[buf], acc_s.at[di_v.at[0, 0]], sem_s).wait()

        for b in range(nbuf):
            fire_gather(b, b)

        def body(ti, carry):
            nxt = ti + nbuf - 1

            @pl.when(jnp.logical_and(lax.rem(nxt, wchunks) == 0,
                                     nxt < nchunk_w))
            def _():
                w1 = nxt // wchunks
                p1 = lax.rem(w1, 2)
                pltpu.sync_copy(src_hbm.at[wbase + w1], si_v.at[p1])
                pltpu.sync_copy(dst_hbm.at[wbase + w1], di_v.at[p1])

            b = lax.rem(ti, nbuf)
            wait_gather(b)
            fire_scatter(ti, b)

            @pl.when(ti >= 1)
            def _():
                bp = lax.rem(ti - 1, nbuf)
                wait_scatter(bp)

                @pl.when(ti - 1 + nbuf < nchunk_w)
                def _():
                    fire_gather(ti - 1 + nbuf, bp)

            return carry

        lax.fori_loop(0, nchunk_w, body, None)
        wait_scatter(lax.rem(nchunk_w - 1, nbuf))
        plsc.subcore_barrier()
        pltpu.sync_copy(
            acc_s.at[pl.ds(s * rows_per_sub, rows_per_sub)],
            out_hbm.at[c, pl.ds(s * rows_per_sub, rows_per_sub)],
        )

    return deg_kernel, prop_kernel


def _mm1_body(x_ref, w_ref, d0_ref, d1_ref, hs_ref, dinv_ref):
    dinv = lax.rsqrt(d0_ref[...] + d1_ref[...] + 1.0)
    h = jnp.dot(x_ref[...], w_ref[...], preferred_element_type=jnp.float32)
    hs_ref[...] = h * dinv
    dinv_ref[...] = dinv


def _mm2_body(p_ref, hs1_ref, dinv_ref, b0_ref, w1_ref, hs2_ref):
    dinv = dinv_ref[...]
    rep = jnp.maximum(
        dinv * (p_ref[0] + p_ref[1] + hs1_ref[...]) + b0_ref[...], 0.0)
    hs2_ref[...] = jnp.dot(
        rep, w1_ref[...], preferred_element_type=jnp.float32) * dinv


def _heads_body(q_ref, hs2_ref, dinv_ref, b1_ref, t_ref,
                w00_ref, b00_ref, w10_ref, b10_ref,
                w01_ref, b01_ref, w11_ref, b11_ref,
                wpp_ref, bpp_ref, p_out, y_out):
    dinv = dinv_ref[...]
    rep = jnp.maximum(
        dinv * (q_ref[0] + q_ref[1] + hs2_ref[...]) + b1_ref[...], 0.0)
    y00 = jnp.maximum(
        jnp.dot(rep, w00_ref[...], preferred_element_type=jnp.float32)
        + b00_ref[...], 0.0)
    y10 = jnp.maximum(
        jnp.dot(rep, w10_ref[...], preferred_element_type=jnp.float32)
        + b10_ref[...], 0.0)
    y0 = jnp.dot(y00, w01_ref[...], preferred_element_type=jnp.float32) \
        + b01_ref[0, 0]
    y1 = jnp.dot(y10, w11_ref[...], preferred_element_type=jnp.float32) \
        + b11_ref[0, 0]
    y_out[...] = jnp.where(t_ref[...] > 0, y1, y0)
    p_out[...] = jax.nn.sigmoid(
        jnp.dot(rep, wpp_ref[...], preferred_element_type=jnp.float32)
        + bpp_ref[0, 0])


def kernel(x, t, z, edge_index, W_gc0, b_gc0, W_gc1, b_gc1,
           W_t00_0, b_t00_0, W_t00_1, b_t00_1,
           W_t10_0, b_t10_0, W_t10_1, b_t10_1,
           W_t01, b_t01, W_t11, b_t11, W_pp, b_pp):
    n, d = x.shape
    e = edge_index.shape[1]

    rows_unit = NS * CHUNK
    n_pad = ((n + 1 + rows_unit - 1) // rows_unit) * rows_unit
    edges_unit = 2 * NC * NS * CHUNK
    e_pad = ((e + edges_unit - 1) // edges_unit) * edges_unit

    n_workers = NC * NS
    edges_per_worker = e_pad // n_workers
    n_chunks = edges_per_worker // CHUNK
    pchunk = 128
    wchunks = 4
    w_sum = e_pad // (pchunk * wchunks * NS)
    q0 = max(1, int(round(w_sum * 0.6)))
    q1 = w_sum - q0

    src_flat = jnp.concatenate(
        [edge_index[0], jnp.zeros((e_pad - e,), jnp.int32)])
    dst_flat = jnp.concatenate(
        [edge_index[1], jnp.full((e_pad - e,), n, jnp.int32)])
    n_windows = e_pad // (pchunk * wchunks)
    src_prop = src_flat.reshape(n_windows, wchunks, pchunk)
    dst_prop = dst_flat.reshape(n_windows, wchunks, pchunk)
    dst_deg = dst_flat.reshape(n_workers, n_chunks, CHUNK)
    x_p = jnp.pad(x, ((0, n_pad - n), (0, 0)))
    t_p = jnp.pad(t, (0, n_pad - n)).reshape(n_pad, 1)

    deg_kernel, prop_kernel = _make_sc_kernels(n_pad, e_pad, d, q0, q1)

    deg = deg_kernel(dst_deg)
    d0 = deg[0].reshape(n_pad, 1)
    d1 = deg[1].reshape(n_pad, 1)

    blk = 1024
    grid = (n_pad // blk,)
    row_spec = pl.BlockSpec((blk, d), lambda i: (i, 0))
    col_spec = pl.BlockSpec((blk, 1), lambda i: (i, 0))
    mat_spec = pl.BlockSpec((d, d), lambda i: (0, 0))
    bias_spec = pl.BlockSpec((1, d), lambda i: (0, 0))
    scal_spec = pl.BlockSpec((1, 1), lambda i: (0, 0))
    part_spec = pl.BlockSpec((2, blk, d), lambda i: (0, i, 0))
    vec_shape = jax.ShapeDtypeStruct((n_pad, d), jnp.float32)
    col_shape = jax.ShapeDtypeStruct((n_pad, 1), jnp.float32)

    hs1, dinv = pl.pallas_call(
        _mm1_body,
        grid=grid,
        in_specs=[row_spec, mat_spec, col_spec, col_spec],
        out_specs=[row_spec, col_spec],
        out_shape=[vec_shape, col_shape],
    )(x_p, W_gc0, d0, d1)

    p_parts = prop_kernel(hs1, src_prop, dst_prop)

    hs2 = pl.pallas_call(
        _mm2_body,
        grid=grid,
        in_specs=[part_spec, row_spec, col_spec, bias_spec, mat_spec],
        out_specs=row_spec,
        out_shape=vec_shape,
    )(p_parts, hs1, dinv, b_gc0.reshape(1, d), W_gc1)

    q_parts = prop_kernel(hs2, src_prop, dst_prop)

    one_spec = pl.BlockSpec((d, 1), lambda i: (0, 0))
    p1, y = pl.pallas_call(
        _heads_body,
        grid=grid,
        in_specs=[part_spec, row_spec, col_spec, bias_spec, col_spec,
                  mat_spec, bias_spec, mat_spec, bias_spec,
                  one_spec, scal_spec, one_spec, scal_spec,
                  one_spec, scal_spec],
        out_specs=[col_spec, col_spec],
        out_shape=[col_shape, col_shape],
    )(q_parts, hs2, dinv, b_gc1.reshape(1, d), t_p,
      W_t00_1, b_t00_1.reshape(1, d), W_t10_1, b_t10_1.reshape(1, d),
      W_t01, b_t01.reshape(1, 1), W_t11, b_t11.reshape(1, 1),
      W_pp, b_pp.reshape(1, 1))

    return (p1[:n], y[:n])


# speedup vs baseline: 1.0638x; 1.0638x over previous
import functools

import jax
import jax.numpy as jnp
from jax import lax
from jax.experimental import pallas as pl
from jax.experimental.pallas import tpu as pltpu
from jax.experimental.pallas import tpu_sc as plsc

NC = 2
NS = 16
LANES = 16
CHUNK = 128


def _fill_vmem_zeros(ref, rows, cols):
    per_row = cols // LANES

    def body(i, carry):
        r = i // per_row
        j = i % per_row
        ref[r, pl.ds(j * LANES, LANES)] = jnp.zeros((LANES,), jnp.float32)
        return carry

    lax.fori_loop(0, rows * per_row, body, None)


@functools.lru_cache(maxsize=None)
def _make_sc_kernels(n_pad, e_pad, d, q0, q1):
    rows_per_sub = n_pad // NS
    edges_per_worker = e_pad // (NC * NS)
    n_chunks = edges_per_worker // CHUNK
    mesh = plsc.VectorSubcoreMesh(core_axis_name="c", subcore_axis_name="s")

    @functools.partial(
        pl.kernel,
        out_type=jax.ShapeDtypeStruct((NC, n_pad), jnp.float32),
        mesh=mesh,
        scratch_types=[
            pltpu.VMEM((n_chunks, CHUNK), jnp.int32),
            pltpu.VMEM((CHUNK,), jnp.float32),
            pltpu.VMEM((rows_per_sub,), jnp.float32),
            pltpu.VMEM_SHARED((n_pad,), jnp.float32),
            pltpu.SemaphoreType.DMA,
        ],
    )
    def deg_kernel(dst_hbm, out_hbm, idx_v, ones_v, zbuf_v, acc_s, sem):
        c = lax.axis_index("c")
        s = lax.axis_index("s")
        wid = c * NS + s

        def fill_ones(i, carry):
            ones_v[pl.ds(i * LANES, LANES)] = jnp.ones((LANES,), jnp.float32)
            return carry

        lax.fori_loop(0, CHUNK // LANES, fill_ones, None)

        def fill_zeros(i, carry):
            zbuf_v[pl.ds(i * LANES, LANES)] = jnp.zeros((LANES,), jnp.float32)
            return carry

        lax.fori_loop(0, rows_per_sub // LANES, fill_zeros, None)
        pltpu.sync_copy(zbuf_v, acc_s.at[pl.ds(s * rows_per_sub, rows_per_sub)])
        pltpu.sync_copy(dst_hbm.at[wid], idx_v)
        plsc.subcore_barrier()

        kgrp = 8

        def body(g, carry):
            descs = [
                pltpu.async_copy(
                    ones_v, acc_s.at[idx_v.at[g * kgrp + b]], sem, add=True)
                for b in range(kgrp)
            ]
            for dsc in descs:
                dsc.wait()
            return carry

        lax.fori_loop(0, n_chunks // kgrp, body, None)
        plsc.subcore_barrier()
        pltpu.sync_copy(
            acc_s.at[pl.ds(s * rows_per_sub, rows_per_sub)],
            out_hbm.at[c, pl.ds(s * rows_per_sub, rows_per_sub)],
        )

    pchunk = 128
    wchunks = 4
    nbuf = 2

    @functools.partial(
        pl.kernel,
        out_type=jax.ShapeDtypeStruct((NC, n_pad, d), jnp.float32),
        mesh=mesh,
        scratch_types=[
            pltpu.VMEM((2, wchunks, pchunk), jnp.int32),
            pltpu.VMEM((2, wchunks, pchunk), jnp.int32),
            pltpu.VMEM((nbuf, pchunk, d), jnp.float32),
            pltpu.VMEM_SHARED((n_pad, d), jnp.float32),
            pltpu.SemaphoreType.DMA,
            pltpu.SemaphoreType.DMA,
        ],
    )
    def prop_kernel(hs_hbm, src_hbm, dst_hbm, out_hbm,
                    si_v, di_v, rows_v, acc_s, sem_g, sem_s):
        c = lax.axis_index("c")
        s = lax.axis_index("s")
        qw = jnp.where(c == 0, q0, q1)
        wbase = c * (NS * q0) + s * qw
        nchunk_w = qw * wchunks

        _fill_vmem_zeros(rows_v.at[0], pchunk, d)

        def zero_acc(k, carry):
            pltpu.sync_copy(
                rows_v.at[0],
                acc_s.at[pl.ds(s * rows_per_sub + k * pchunk, pchunk)])
            return carry

        lax.fori_loop(0, rows_per_sub // pchunk, zero_acc, None)
        pltpu.sync_copy(src_hbm.at[wbase], si_v.at[0])
        pltpu.sync_copy(dst_hbm.at[wbase], di_v.at[0])
        plsc.subcore_barrier()

        def fire_gather(ti, buf):
            pltpu.async_copy(
                hs_hbm.at[si_v.at[lax.rem(ti // wchunks, 2),
                                  lax.rem(ti, wchunks)]],
                rows_v.at[buf], sem_g)

        def wait_gather(buf):
            pltpu.make_async_copy(
                hs_hbm.at[si_v.at[0, 0]], rows_v.at[buf], sem_g).wait()

        def fire_scatter(ti, buf):
            pltpu.async_copy(
                rows_v.at[buf],
                acc_s.at[di_v.at[lax.rem(ti // wchunks, 2),
                                 lax.rem(ti, wchunks)]],
                sem_s, add=True)

        def wait_scatter(buf):
            pltpu.make_async_copy(
                rows_v.at[buf], acc_s.at[di_v.at[0, 0]], sem_s).wait()

        for b in range(nbuf):
            fire_gather(b, b)

        def body(ti, carry):
            nxt = ti + nbuf - 1

            @pl.when(jnp.logical_and(lax.rem(nxt, wchunks) == 0,
                                     nxt < nchunk_w))
            def _():
                w1 = nxt // wchunks
                p1 = lax.rem(w1, 2)
                pltpu.sync_copy(src_hbm.at[wbase + w1], si_v.at[p1])
                pltpu.sync_copy(dst_hbm.at[wbase + w1], di_v.at[p1])

            b = lax.rem(ti, nbuf)
            wait_gather(b)
            fire_scatter(ti, b)

            @pl.when(ti >= 1)
            def _():
                bp = lax.rem(ti - 1, nbuf)
                wait_scatter(bp)

                @pl.when(ti - 1 + nbuf < nchunk_w)
                def _():
                    fire_gather(ti - 1 + nbuf, bp)

            return carry

        lax.fori_loop(0, nchunk_w, body, None)
        wait_scatter(lax.rem(nchunk_w - 1, nbuf))
        plsc.subcore_barrier()
        pltpu.sync_copy(
            acc_s.at[pl.ds(s * rows_per_sub, rows_per_sub)],
            out_hbm.at[c, pl.ds(s * rows_per_sub, rows_per_sub)],
        )

    return deg_kernel, prop_kernel


def _mm1_body(x_ref, w_ref, d0_ref, d1_ref, hs_ref, dinv_ref):
    dinv = lax.rsqrt(d0_ref[...] + d1_ref[...] + 1.0)
    h = jnp.dot(x_ref[...], w_ref[...], preferred_element_type=jnp.float32)
    hs_ref[...] = h * dinv
    dinv_ref[...] = dinv


def _mm2_body(p_ref, hs1_ref, dinv_ref, b0_ref, w1_ref, hs2_ref):
    dinv = dinv_ref[...]
    rep = jnp.maximum(
        dinv * (p_ref[0] + p_ref[1] + hs1_ref[...]) + b0_ref[...], 0.0)
    hs2_ref[...] = jnp.dot(
        rep, w1_ref[...], preferred_element_type=jnp.float32) * dinv


def _heads_body(q_ref, hs2_ref, dinv_ref, b1_ref, t_ref,
                w00_ref, b00_ref, w10_ref, b10_ref,
                w01_ref, b01_ref, w11_ref, b11_ref,
                wpp_ref, bpp_ref, p_out, y_out):
    dinv = dinv_ref[...]
    rep = jnp.maximum(
        dinv * (q_ref[0] + q_ref[1] + hs2_ref[...]) + b1_ref[...], 0.0)
    y00 = jnp.maximum(
        jnp.dot(rep, w00_ref[...], preferred_element_type=jnp.float32)
        + b00_ref[...], 0.0)
    y10 = jnp.maximum(
        jnp.dot(rep, w10_ref[...], preferred_element_type=jnp.float32)
        + b10_ref[...], 0.0)
    y0 = jnp.dot(y00, w01_ref[...], preferred_element_type=jnp.float32) \
        + b01_ref[0, 0]
    y1 = jnp.dot(y10, w11_ref[...], preferred_element_type=jnp.float32) \
        + b11_ref[0, 0]
    y_out[...] = jnp.where(t_ref[...] > 0, y1, y0)
    p_out[...] = jax.nn.sigmoid(
        jnp.dot(rep, wpp_ref[...], preferred_element_type=jnp.float32)
        + bpp_ref[0, 0])


def kernel(x, t, z, edge_index, W_gc0, b_gc0, W_gc1, b_gc1,
           W_t00_0, b_t00_0, W_t00_1, b_t00_1,
           W_t10_0, b_t10_0, W_t10_1, b_t10_1,
           W_t01, b_t01, W_t11, b_t11, W_pp, b_pp):
    n, d = x.shape
    e = edge_index.shape[1]

    rows_unit = NS * CHUNK
    n_pad = ((n + 1 + rows_unit - 1) // rows_unit) * rows_unit
    edges_unit = 2 * NC * NS * CHUNK
    e_pad = ((e + edges_unit - 1) // edges_unit) * edges_unit

    n_workers = NC * NS
    edges_per_worker = e_pad // n_workers
    n_chunks = edges_per_worker // CHUNK
    pchunk = 128
    wchunks = 4
    w_sum = e_pad // (pchunk * wchunks * NS)
    q0 = max(1, int(round(w_sum * 0.8)))
    q1 = w_sum - q0

    src_flat = jnp.concatenate(
        [edge_index[0], jnp.zeros((e_pad - e,), jnp.int32)])
    dst_flat = jnp.concatenate(
        [edge_index[1], jnp.full((e_pad - e,), n, jnp.int32)])
    n_windows = e_pad // (pchunk * wchunks)
    src_prop = src_flat.reshape(n_windows, wchunks, pchunk)
    dst_prop = dst_flat.reshape(n_windows, wchunks, pchunk)
    dst_deg = dst_flat.reshape(n_workers, n_chunks, CHUNK)
    x_p = jnp.pad(x, ((0, n_pad - n), (0, 0)))
    t_p = jnp.pad(t, (0, n_pad - n)).reshape(n_pad, 1)

    deg_kernel, prop_kernel = _make_sc_kernels(n_pad, e_pad, d, q0, q1)

    deg = deg_kernel(dst_deg)
    d0 = deg[0].reshape(n_pad, 1)
    d1 = deg[1].reshape(n_pad, 1)

    blk = 1024
    grid = (n_pad // blk,)
    row_spec = pl.BlockSpec((blk, d), lambda i: (i, 0))
    col_spec = pl.BlockSpec((blk, 1), lambda i: (i, 0))
    mat_spec = pl.BlockSpec((d, d), lambda i: (0, 0))
    bias_spec = pl.BlockSpec((1, d), lambda i: (0, 0))
    scal_spec = pl.BlockSpec((1, 1), lambda i: (0, 0))
    part_spec = pl.BlockSpec((2, blk, d), lambda i: (0, i, 0))
    vec_shape = jax.ShapeDtypeStruct((n_pad, d), jnp.float32)
    col_shape = jax.ShapeDtypeStruct((n_pad, 1), jnp.float32)

    hs1, dinv = pl.pallas_call(
        _mm1_body,
        grid=grid,
        in_specs=[row_spec, mat_spec, col_spec, col_spec],
        out_specs=[row_spec, col_spec],
        out_shape=[vec_shape, col_shape],
    )(x_p, W_gc0, d0, d1)

    p_parts = prop_kernel(hs1, src_prop, dst_prop)

    hs2 = pl.pallas_call(
        _mm2_body,
        grid=grid,
        in_specs=[part_spec, row_spec, col_spec, bias_spec, mat_spec],
        out_specs=row_spec,
        out_shape=vec_shape,
    )(p_parts, hs1, dinv, b_gc0.reshape(1, d), W_gc1)

    q_parts = prop_kernel(hs2, src_prop, dst_prop)

    one_spec = pl.BlockSpec((d, 1), lambda i: (0, 0))
    p1, y = pl.pallas_call(
        _heads_body,
        grid=grid,
        in_specs=[part_spec, row_spec, col_spec, bias_spec, col_spec,
                  mat_spec, bias_spec, mat_spec, bias_spec,
                  one_spec, scal_spec, one_spec, scal_spec,
                  one_spec, scal_spec],
        out_specs=[col_spec, col_spec],
        out_shape=[col_shape, col_shape],
    )(q_parts, hs2, dinv, b_gc1.reshape(1, d), t_p,
      W_t00_1, b_t00_1.reshape(1, d), W_t10_1, b_t10_1.reshape(1, d),
      W_t01, b_t01.reshape(1, 1), W_t11, b_t11.reshape(1, 1),
      W_pp, b_pp.reshape(1, 1))

    return (p1[:n], y[:n])


# speedup vs baseline: 1.0971x; 1.0313x over previous
import functools

import jax
import jax.numpy as jnp
from jax import lax
from jax.experimental import pallas as pl
from jax.experimental.pallas import tpu as pltpu
from jax.experimental.pallas import tpu_sc as plsc

NC = 2
NS = 16
LANES = 16
CHUNK = 128


def _fill_vmem_zeros(ref, rows, cols):
    per_row = cols // LANES

    def body(i, carry):
        r = i // per_row
        j = i % per_row
        ref[r, pl.ds(j * LANES, LANES)] = jnp.zeros((LANES,), jnp.float32)
        return carry

    lax.fori_loop(0, rows * per_row, body, None)


@functools.lru_cache(maxsize=None)
def _make_sc_kernels(n_pad, e_pad, d, q0, q1):
    rows_per_sub = n_pad // NS
    edges_per_worker = e_pad // (NC * NS)
    n_chunks = edges_per_worker // CHUNK
    mesh = plsc.VectorSubcoreMesh(core_axis_name="c", subcore_axis_name="s")

    @functools.partial(
        pl.kernel,
        out_type=jax.ShapeDtypeStruct((NC, n_pad), jnp.float32),
        mesh=mesh,
        scratch_types=[
            pltpu.VMEM((n_chunks, CHUNK), jnp.int32),
            pltpu.VMEM((CHUNK,), jnp.float32),
            pltpu.VMEM((rows_per_sub,), jnp.float32),
            pltpu.VMEM_SHARED((n_pad,), jnp.float32),
            pltpu.SemaphoreType.DMA,
        ],
    )
    def deg_kernel(dst_hbm, out_hbm, idx_v, ones_v, zbuf_v, acc_s, sem):
        c = lax.axis_index("c")
        s = lax.axis_index("s")
        wid = c * NS + s

        def fill_ones(i, carry):
            ones_v[pl.ds(i * LANES, LANES)] = jnp.ones((LANES,), jnp.float32)
            return carry

        lax.fori_loop(0, CHUNK // LANES, fill_ones, None)

        def fill_zeros(i, carry):
            zbuf_v[pl.ds(i * LANES, LANES)] = jnp.zeros((LANES,), jnp.float32)
            return carry

        lax.fori_loop(0, rows_per_sub // LANES, fill_zeros, None)
        pltpu.sync_copy(zbuf_v, acc_s.at[pl.ds(s * rows_per_sub, rows_per_sub)])
        pltpu.sync_copy(dst_hbm.at[wid], idx_v)
        plsc.subcore_barrier()

        kgrp = 8

        def body(g, carry):
            descs = [
                pltpu.async_copy(
                    ones_v, acc_s.at[idx_v.at[g * kgrp + b]], sem, add=True)
                for b in range(kgrp)
            ]
            for dsc in descs:
                dsc.wait()
            return carry

        lax.fori_loop(0, n_chunks // kgrp, body, None)
        plsc.subcore_barrier()
        pltpu.sync_copy(
            acc_s.at[pl.ds(s * rows_per_sub, rows_per_sub)],
            out_hbm.at[c, pl.ds(s * rows_per_sub, rows_per_sub)],
        )

    pchunk = 128
    wchunks = 4
    nbuf = 2

    @functools.partial(
        pl.kernel,
        out_type=jax.ShapeDtypeStruct((NC, n_pad, d), jnp.float32),
        mesh=mesh,
        scratch_types=[
            pltpu.VMEM((2, wchunks, pchunk), jnp.int32),
            pltpu.VMEM((2, wchunks, pchunk), jnp.int32),
            pltpu.VMEM((nbuf, pchunk, d), jnp.float32),
            pltpu.VMEM_SHARED((n_pad, d), jnp.float32),
            pltpu.SemaphoreType.DMA,
            pltpu.SemaphoreType.DMA,
        ],
    )
    def prop_kernel(hs_hbm, src_hbm, dst_hbm, out_hbm,
                    si_v, di_v, rows_v, acc_s, sem_g, sem_s):
        c = lax.axis_index("c")
        s = lax.axis_index("s")
        qw = jnp.where(c == 0, q0, q1)
        wbase = c * (NS * q0) + s * qw
        nchunk_w = qw * wchunks

        _fill_vmem_zeros(rows_v.at[0], pchunk, d)

        def zero_acc(k, carry):
            pltpu.sync_copy(
                rows_v.at[0],
                acc_s.at[pl.ds(s * rows_per_sub + k * pchunk, pchunk)])
            return carry

        lax.fori_loop(0, rows_per_sub // pchunk, zero_acc, None)
        pltpu.sync_copy(src_hbm.at[wbase], si_v.at[0])
        pltpu.sync_copy(dst_hbm.at[wbase], di_v.at[0])
        plsc.subcore_barrier()

        def fire_gather(ti, buf):
            pltpu.async_copy(
                hs_hbm.at[si_v.at[lax.rem(ti // wchunks, 2),
                                  lax.rem(ti, wchunks)]],
                rows_v.at[buf], sem_g)

        def wait_gather(buf):
            pltpu.make_async_copy(
                hs_hbm.at[si_v.at[0, 0]], rows_v.at[buf], sem_g).wait()

        def fire_scatter(ti, buf):
            pltpu.async_copy(
                rows_v.at[buf],
                acc_s.at[di_v.at[lax.rem(ti // wchunks, 2),
                                 lax.rem(ti, wchunks)]],
                sem_s, add=True)

        def wait_scatter(buf):
            pltpu.make_async_copy(
                rows_v.at[buf], acc_s.at[di_v.at[0, 0]], sem_s).wait()

        for b in range(nbuf):
            fire_gather(b, b)

        def body(ti, carry):
            nxt = ti + nbuf - 1

            @pl.when(jnp.logical_and(lax.rem(nxt, wchunks) == 0,
                                     nxt < nchunk_w))
            def _():
                w1 = nxt // wchunks
                p1 = lax.rem(w1, 2)
                pltpu.sync_copy(src_hbm.at[wbase + w1], si_v.at[p1])
                pltpu.sync_copy(dst_hbm.at[wbase + w1], di_v.at[p1])

            b = lax.rem(ti, nbuf)
            wait_gather(b)
            fire_scatter(ti, b)

            @pl.when(ti >= 1)
            def _():
                bp = lax.rem(ti - 1, nbuf)
                wait_scatter(bp)

                @pl.when(ti - 1 + nbuf < nchunk_w)
                def _():
                    fire_gather(ti - 1 + nbuf, bp)

            return carry

        lax.fori_loop(0, nchunk_w, body, None)
        wait_scatter(lax.rem(nchunk_w - 1, nbuf))
        plsc.subcore_barrier()
        pltpu.sync_copy(
            acc_s.at[pl.ds(s * rows_per_sub, rows_per_sub)],
            out_hbm.at[c, pl.ds(s * rows_per_sub, rows_per_sub)],
        )

    return deg_kernel, prop_kernel


def _mm1_body(x_ref, w_ref, d0_ref, d1_ref, hs_ref, dinv_ref):
    dinv = lax.rsqrt(d0_ref[...] + d1_ref[...] + 1.0)
    h = jnp.dot(x_ref[...], w_ref[...], preferred_element_type=jnp.float32)
    hs_ref[...] = h * dinv
    dinv_ref[...] = dinv


def _mm2_body(p_ref, hs1_ref, dinv_ref, b0_ref, w1_ref, hs2_ref):
    dinv = dinv_ref[...]
    rep = jnp.maximum(
        dinv * (p_ref[0] + p_ref[1] + hs1_ref[...]) + b0_ref[...], 0.0)
    hs2_ref[...] = jnp.dot(
        rep, w1_ref[...], preferred_element_type=jnp.float32) * dinv


def _heads_body(q_ref, hs2_ref, dinv_ref, b1_ref, t_ref,
                w00_ref, b00_ref, w10_ref, b10_ref,
                w01_ref, b01_ref, w11_ref, b11_ref,
                wpp_ref, bpp_ref, p_out, y_out):
    dinv = dinv_ref[...]
    rep = jnp.maximum(
        dinv * (q_ref[0] + q_ref[1] + hs2_ref[...]) + b1_ref[...], 0.0)
    y00 = jnp.maximum(
        jnp.dot(rep, w00_ref[...], preferred_element_type=jnp.float32)
        + b00_ref[...], 0.0)
    y10 = jnp.maximum(
        jnp.dot(rep, w10_ref[...], preferred_element_type=jnp.float32)
        + b10_ref[...], 0.0)
    y0 = jnp.dot(y00, w01_ref[...], preferred_element_type=jnp.float32) \
        + b01_ref[0, 0]
    y1 = jnp.dot(y10, w11_ref[...], preferred_element_type=jnp.float32) \
        + b11_ref[0, 0]
    y_out[...] = jnp.where(t_ref[...] > 0, y1, y0)
    p_out[...] = jax.nn.sigmoid(
        jnp.dot(rep, wpp_ref[...], preferred_element_type=jnp.float32)
        + bpp_ref[0, 0])


def kernel(x, t, z, edge_index, W_gc0, b_gc0, W_gc1, b_gc1,
           W_t00_0, b_t00_0, W_t00_1, b_t00_1,
           W_t10_0, b_t10_0, W_t10_1, b_t10_1,
           W_t01, b_t01, W_t11, b_t11, W_pp, b_pp):
    n, d = x.shape
    e = edge_index.shape[1]

    rows_unit = NS * CHUNK
    n_pad = ((n + 1 + rows_unit - 1) // rows_unit) * rows_unit
    edges_unit = 2 * NC * NS * CHUNK
    e_pad = ((e + edges_unit - 1) // edges_unit) * edges_unit

    n_workers = NC * NS
    edges_per_worker = e_pad // n_workers
    n_chunks = edges_per_worker // CHUNK
    pchunk = 128
    wchunks = 4
    w_sum = e_pad // (pchunk * wchunks * NS)
    q0 = max(1, int(round(w_sum * 0.9)))
    q1 = w_sum - q0

    src_flat = jnp.concatenate(
        [edge_index[0], jnp.zeros((e_pad - e,), jnp.int32)])
    dst_flat = jnp.concatenate(
        [edge_index[1], jnp.full((e_pad - e,), n, jnp.int32)])
    n_windows = e_pad // (pchunk * wchunks)
    src_prop = src_flat.reshape(n_windows, wchunks, pchunk)
    dst_prop = dst_flat.reshape(n_windows, wchunks, pchunk)
    dst_deg = dst_flat.reshape(n_workers, n_chunks, CHUNK)
    x_p = jnp.pad(x, ((0, n_pad - n), (0, 0)))
    t_p = jnp.pad(t, (0, n_pad - n)).reshape(n_pad, 1)

    deg_kernel, prop_kernel = _make_sc_kernels(n_pad, e_pad, d, q0, q1)

    deg = deg_kernel(dst_deg)
    d0 = deg[0].reshape(n_pad, 1)
    d1 = deg[1].reshape(n_pad, 1)

    blk = 1024
    grid = (n_pad // blk,)
    row_spec = pl.BlockSpec((blk, d), lambda i: (i, 0))
    col_spec = pl.BlockSpec((blk, 1), lambda i: (i, 0))
    mat_spec = pl.BlockSpec((d, d), lambda i: (0, 0))
    bias_spec = pl.BlockSpec((1, d), lambda i: (0, 0))
    scal_spec = pl.BlockSpec((1, 1), lambda i: (0, 0))
    part_spec = pl.BlockSpec((2, blk, d), lambda i: (0, i, 0))
    vec_shape = jax.ShapeDtypeStruct((n_pad, d), jnp.float32)
    col_shape = jax.ShapeDtypeStruct((n_pad, 1), jnp.float32)

    hs1, dinv = pl.pallas_call(
        _mm1_body,
        grid=grid,
        in_specs=[row_spec, mat_spec, col_spec, col_spec],
        out_specs=[row_spec, col_spec],
        out_shape=[vec_shape, col_shape],
    )(x_p, W_gc0, d0, d1)

    p_parts = prop_kernel(hs1, src_prop, dst_prop)

    hs2 = pl.pallas_call(
        _mm2_body,
        grid=grid,
        in_specs=[part_spec, row_spec, col_spec, bias_spec, mat_spec],
        out_specs=row_spec,
        out_shape=vec_shape,
    )(p_parts, hs1, dinv, b_gc0.reshape(1, d), W_gc1)

    q_parts = prop_kernel(hs2, src_prop, dst_prop)

    one_spec = pl.BlockSpec((d, 1), lambda i: (0, 0))
    p1, y = pl.pallas_call(
        _heads_body,
        grid=grid,
        in_specs=[part_spec, row_spec, col_spec, bias_spec, col_spec,
                  mat_spec, bias_spec, mat_spec, bias_spec,
                  one_spec, scal_spec, one_spec, scal_spec,
                  one_spec, scal_spec],
        out_specs=[col_spec, col_spec],
        out_shape=[col_shape, col_shape],
    )(q_parts, hs2, dinv, b_gc1.reshape(1, d), t_p,
      W_t00_1, b_t00_1.reshape(1, d), W_t10_1, b_t10_1.reshape(1, d),
      W_t01, b_t01.reshape(1, 1), W_t11, b_t11.reshape(1, 1),
      W_pp, b_pp.reshape(1, 1))

    return (p1[:n], y[:n])
